# baseline (speedup 1.0000x reference)
import functools

import jax
import jax.numpy as jnp
from jax import lax
from jax.experimental import pallas as pl
from jax.experimental.pallas import tpu as pltpu
from jax.experimental.pallas import tpu_sc as plsc

L = 256
D = 32768
NC = 2
NS = 16
NW = NC * NS
RPW = L // NW

NCH = 4
CHW = D // NCH


def _permute_body(x_hbm, perm_hbm, out_hbm, pvm, bufs, gsems, ssems,
                  cgsems, cssems):
    c = lax.axis_index("c")
    s = lax.axis_index("s")
    wid = s * NC + c
    base = wid * RPW

    pltpu.sync_copy(perm_hbm.at[pl.ds(base, RPW)], pvm.at[pl.ds(0, RPW)])
    vals = pvm[...]

    def gather_row(k, buf, sem):
        return pltpu.async_copy(x_hbm.at[pl.ds(vals[k], 1)], buf, sem)

    def store_row(k, buf, sem):
        return pltpu.async_copy(buf, out_hbm.at[pl.ds(base + k, 1)], sem)

    def gather_chunk(k, buf, ci, sem):
        return pltpu.async_copy(
            x_hbm.at[pl.ds(vals[k], 1), pl.ds(ci * CHW, CHW)],
            buf.at[:, pl.ds(ci * CHW, CHW)], sem)

    def store_chunk(k, buf, ci, sem):
        return pltpu.async_copy(
            buf.at[:, pl.ds(ci * CHW, CHW)],
            out_hbm.at[pl.ds(base + k, 1), pl.ds(ci * CHW, CHW)], sem)

    A, B, C = bufs

    gc0 = [gather_chunk(0, A, ci, cgsems[ci]) for ci in range(NCH)]
    g1 = gather_row(1, B, gsems[1])
    g2 = gather_row(2, C, gsems[2])
    sc0 = []
    for ci in range(NCH):
        gc0[ci].wait()
        sc0.append(store_chunk(0, A, ci, cssems[ci]))

    g1.wait()
    st1 = store_row(1, B, ssems[1])
    for cp in sc0:
        cp.wait()
    g3 = gather_row(3, A, gsems[0])

    g2.wait()
    st2 = store_row(2, C, ssems[2])
    st1.wait()
    g4 = gather_row(4, B, gsems[1])

    g3.wait()
    st3 = store_row(3, A, ssems[0])
    st2.wait()
    g5 = gather_row(5, C, gsems[2])

    g4.wait()
    st4 = store_row(4, B, ssems[1])
    st3.wait()
    g6 = gather_row(6, A, gsems[0])

    g5.wait()
    st5 = store_row(5, C, ssems[2])
    st4.wait()
    gc7 = [gather_chunk(7, B, ci, cgsems[ci]) for ci in range(NCH)]

    g6.wait()
    st6 = store_row(6, A, ssems[0])
    sc7 = []
    for ci in range(NCH):
        gc7[ci].wait()
        sc7.append(store_chunk(7, B, ci, cssems[ci]))

    st5.wait()
    st6.wait()
    for cp in sc7:
        cp.wait()


@functools.partial(
    pl.kernel,
    out_type=jax.ShapeDtypeStruct((L, D), jnp.float32),
    mesh=plsc.VectorSubcoreMesh(core_axis_name="c", subcore_axis_name="s"),
    scratch_types=[
        pltpu.VMEM((16,), jnp.int32),
        [pltpu.VMEM((1, D), jnp.float32)] * 3,
        [pltpu.SemaphoreType.DMA] * 3,
        [pltpu.SemaphoreType.DMA] * 3,
        [pltpu.SemaphoreType.DMA] * NCH,
        [pltpu.SemaphoreType.DMA] * NCH,
    ],
)
def _permute(x_hbm, perm_hbm, out_hbm, pvm, bufs, gsems, ssems,
             cgsems, cssems):
    _permute_body(x_hbm, perm_hbm, out_hbm, pvm, bufs, gsems, ssems,
                  cgsems, cssems)


def kernel(x, permutations):
    perm1d = permutations.astype(jnp.int32)
    return _permute(x, perm1d)


# speedup vs baseline: 1.0237x; 1.0237x over previous
import functools

import jax
import jax.numpy as jnp
from jax import lax
from jax.experimental import pallas as pl
from jax.experimental.pallas import tpu as pltpu
from jax.experimental.pallas import tpu_sc as plsc

L = 256
D = 32768
NC = 2
NS = 16
NW = NC * NS
RPW = L // NW

NBUF = 3


def _permute_body(x_hbm, perm_hbm, out_hbm, pvm, bufs, gsems, ssems):
    c = lax.axis_index("c")
    s = lax.axis_index("s")
    wid = s * NC + c
    base = wid * RPW

    pltpu.sync_copy(perm_hbm.at[pl.ds(base, RPW)], pvm.at[pl.ds(0, RPW)])
    vals = pvm[...]

    g = [None] * RPW
    st = [None] * RPW
    for k in range(NBUF):
        g[k] = pltpu.async_copy(x_hbm.at[pl.ds(vals[k], 1)], bufs[k],
                                gsems[k])
    for k in range(RPW):
        sl = k % NBUF
        g[k].wait()
        st[k] = pltpu.async_copy(bufs[sl], out_hbm.at[pl.ds(base + k, 1)],
                                 ssems[sl])
        if k + NBUF < RPW:
            st[k].wait()
            g[k + NBUF] = pltpu.async_copy(
                x_hbm.at[pl.ds(vals[k + NBUF], 1)], bufs[sl], gsems[sl])
    for k in range(RPW - NBUF, RPW):
        if st[k] is not None:
            st[k].wait()


@functools.partial(
    pl.kernel,
    out_type=jax.ShapeDtypeStruct((L, D), jnp.float32),
    mesh=plsc.VectorSubcoreMesh(core_axis_name="c", subcore_axis_name="s"),
    scratch_types=[
        pltpu.VMEM((16,), jnp.int32),
        [pltpu.VMEM((1, D), jnp.float32)] * NBUF,
        [pltpu.SemaphoreType.DMA] * NBUF,
        [pltpu.SemaphoreType.DMA] * NBUF,
    ],
)
def _permute(x_hbm, perm_hbm, out_hbm, pvm, bufs, gsems, ssems):
    _permute_body(x_hbm, perm_hbm, out_hbm, pvm, bufs, gsems, ssems)


def kernel(x, permutations):
    perm1d = permutations.astype(jnp.int32)
    return _permute(x, perm1d)
